# Initial kernel scaffold; baseline (speedup 1.0000x reference)
import jax
import jax.numpy as jnp
from jax.experimental import pallas as pl


def kernel(inputs, tables):
    raise NotImplementedError("write your pallas kernel here")


# speedup vs baseline: 13.0559x; 13.0559x over previous
import functools

import jax
import jax.numpy as jnp
from jax import lax
from jax.experimental import pallas as pl
from jax.experimental.pallas import tpu as pltpu
from jax.experimental.pallas import tpu_sc as plsc

B = 16384
F = 26
V = 100
D = 64

NC = 2
NS = 16
NW = NC * NS

ROWS = B * F
RPW = ROWS // NW
CHUNK = 128
NCHUNK = RPW // CHUNK

_mesh = plsc.VectorSubcoreMesh(core_axis_name="c", subcore_axis_name="s")


@functools.partial(
    pl.kernel,
    mesh=_mesh,
    out_type=jax.ShapeDtypeStruct((ROWS, D), jnp.float32),
    compiler_params=pltpu.CompilerParams(use_tc_tiling_on_sc=False),
    scratch_types=[
        pltpu.VMEM((NCHUNK, CHUNK), jnp.int32),
        pltpu.VMEM((2, CHUNK, D), jnp.float32),
        pltpu.SemaphoreType.DMA,
        pltpu.SemaphoreType.DMA,
    ],
)
def _emb_lookup(idx_hbm, tab_hbm, out_hbm, idx_v, rows_v, gsem, wsem):
    cid = lax.axis_index("c")
    sid = lax.axis_index("s")
    wid = sid * NC + cid
    row_base = wid * RPW

    pltpu.sync_copy(idx_hbm.at[pl.ds(wid * NCHUNK, NCHUNK)], idx_v)

    def xform(j, carry):
        for g in range(CHUNK // 16):
            pos = j * CHUNK + g * 16 + lax.iota(jnp.int32, 16)
            f = lax.rem(pos, F)
            sl = pl.ds(g * 16, 16)
            idx_v[j, sl] = idx_v[j, sl] + f * V
        return carry

    lax.fori_loop(0, NCHUNK, xform, 0, unroll=False)

    def fire(c, buf):
        pltpu.async_copy(tab_hbm.at[idx_v.at[c]], rows_v.at[buf], gsem)

    def drain_gather(buf):
        pltpu.make_async_copy(
            tab_hbm.at[pl.ds(0, CHUNK)], rows_v.at[buf], gsem
        ).wait()

    def drain_write(buf):
        pltpu.make_async_copy(
            rows_v.at[buf], tab_hbm.at[pl.ds(0, CHUNK)], wsem
        ).wait()

    fire(0, 0)

    def step(half, carry):
        for b in range(2):
            c = half * 2 + b
            drain_gather(b)

            @pl.when(c >= 1)
            def _():
                drain_write(1 - b)

            @pl.when(c + 1 < NCHUNK)
            def _():
                fire(c + 1, 1 - b)

            pltpu.async_copy(
                rows_v.at[b],
                out_hbm.at[pl.ds(row_base + c * CHUNK, CHUNK)],
                wsem,
            )
        return carry

    lax.fori_loop(0, NCHUNK // 2, step, 0)

    drain_write((NCHUNK - 1) % 2)


def kernel(inputs, tables):
    idx = inputs.reshape(ROWS // CHUNK, CHUNK)
    tab = tables.reshape(F * V, D)
    out = _emb_lookup(idx, tab)
    return out.reshape(B, F, D)


# speedup vs baseline: 14.0822x; 1.0786x over previous
import functools

import jax
import jax.numpy as jnp
from jax import lax
from jax.experimental import pallas as pl
from jax.experimental.pallas import tpu as pltpu
from jax.experimental.pallas import tpu_sc as plsc

B = 16384
F = 26
V = 100
D = 64

NC = 2
NS = 16
NW = NC * NS

ROWS = B * F
RPW = ROWS // NW
CHUNK = 128
NCHUNK = RPW // CHUNK
NBUF = 8
INFLIGHT = 6

_mesh = plsc.VectorSubcoreMesh(core_axis_name="c", subcore_axis_name="s")


@functools.partial(
    pl.kernel,
    mesh=_mesh,
    out_type=jax.ShapeDtypeStruct((ROWS, D), jnp.float32),
    compiler_params=pltpu.CompilerParams(use_tc_tiling_on_sc=False),
    scratch_types=[
        pltpu.VMEM((NCHUNK, CHUNK), jnp.int32),
        pltpu.VMEM((NBUF, CHUNK, D), jnp.float32),
        pltpu.SemaphoreType.DMA,
        pltpu.SemaphoreType.DMA,
    ],
)
def _emb_lookup(idx_hbm, tab_hbm, out_hbm, idx_v, rows_v, gsem, wsem):
    cid = lax.axis_index("c")
    sid = lax.axis_index("s")
    wid = sid * NC + cid
    row_base = wid * RPW

    pltpu.sync_copy(idx_hbm.at[pl.ds(wid * NCHUNK, NCHUNK)], idx_v)

    def xform(j):
        for g in range(CHUNK // 16):
            pos = j * CHUNK + g * 16 + lax.iota(jnp.int32, 16)
            f = lax.rem(pos, F)
            sl = pl.ds(g * 16, 16)
            idx_v[j, sl] = idx_v[j, sl] + f * V

    def fire(c, buf):
        pltpu.async_copy(tab_hbm.at[idx_v.at[c]], rows_v.at[buf], gsem)

    def drain_gather(buf):
        pltpu.make_async_copy(
            tab_hbm.at[pl.ds(0, CHUNK)], rows_v.at[buf], gsem
        ).wait()

    def drain_write(buf):
        pltpu.make_async_copy(
            rows_v.at[buf], tab_hbm.at[pl.ds(0, CHUNK)], wsem
        ).wait()

    for i in range(INFLIGHT):
        xform(i)
        fire(i, i)

    def step(o, carry):
        for b in range(NBUF):
            c = o * NBUF + b
            drain_gather(b)

            @pl.when(c <= NCHUNK - 1 - INFLIGHT)
            def _():
                @pl.when(c >= NBUF - INFLIGHT)
                def _():
                    drain_write((b + INFLIGHT) % NBUF)

                xform(c + INFLIGHT)
                fire(c + INFLIGHT, (b + INFLIGHT) % NBUF)

            pltpu.async_copy(
                rows_v.at[b],
                out_hbm.at[pl.ds(row_base + c * CHUNK, CHUNK)],
                wsem,
            )
        return carry

    lax.fori_loop(0, NCHUNK // NBUF, step, 0)

    for b in range(NBUF):
        drain_write(b)


def kernel(inputs, tables):
    idx = inputs.reshape(ROWS // CHUNK, CHUNK)
    tab = tables.reshape(F * V, D)
    out = _emb_lookup(idx, tab)
    return out.reshape(B, F, D)


# speedup vs baseline: 16.5499x; 1.1752x over previous
import functools

import jax
import jax.numpy as jnp
from jax import lax
from jax.experimental import pallas as pl
from jax.experimental.pallas import tpu as pltpu
from jax.experimental.pallas import tpu_sc as plsc

B = 16384
F = 26
V = 100
D = 64

NC = 2
NS = 16
NW = NC * NS

ROWS = B * F
RPW = ROWS // NW
CHUNK = 128
NCHUNK = RPW // CHUNK
NBUF = 8
INFLIGHT = 6

_mesh = plsc.VectorSubcoreMesh(core_axis_name="c", subcore_axis_name="s")


@functools.partial(
    pl.kernel,
    mesh=_mesh,
    out_type=jax.ShapeDtypeStruct((ROWS, D), jnp.float32),
    compiler_params=pltpu.CompilerParams(use_tc_tiling_on_sc=False),
    scratch_types=[
        pltpu.VMEM((NCHUNK, CHUNK), jnp.int32),
        pltpu.VMEM((NBUF, CHUNK, D), jnp.float32),
        pltpu.VMEM_SHARED((F * V, D), jnp.float32),
        pltpu.SemaphoreType.DMA,
        pltpu.SemaphoreType.DMA,
    ],
)
def _emb_lookup(idx_hbm, tab_hbm, out_hbm, idx_v, rows_v, tab_s, gsem, wsem):
    cid = lax.axis_index("c")
    sid = lax.axis_index("s")
    wid = sid * NC + cid
    row_base = wid * RPW

    @pl.when(sid == 0)
    def _():
        pltpu.sync_copy(tab_hbm, tab_s)

    pltpu.sync_copy(idx_hbm.at[pl.ds(wid * NCHUNK, NCHUNK)], idx_v)
    plsc.subcore_barrier()

    def xform(j):
        for g in range(CHUNK // 16):
            pos = j * CHUNK + g * 16 + lax.iota(jnp.int32, 16)
            f = lax.rem(pos, F)
            sl = pl.ds(g * 16, 16)
            idx_v[j, sl] = idx_v[j, sl] + f * V

    def fire(c, buf):
        pltpu.async_copy(tab_s.at[idx_v.at[c]], rows_v.at[buf], gsem)

    def drain_gather(buf):
        pltpu.make_async_copy(
            tab_hbm.at[pl.ds(0, CHUNK)], rows_v.at[buf], gsem
        ).wait()

    def drain_write(buf):
        pltpu.make_async_copy(
            rows_v.at[buf], tab_hbm.at[pl.ds(0, CHUNK)], wsem
        ).wait()

    for i in range(INFLIGHT):
        xform(i)
        fire(i, i)

    def step(o, carry):
        for b in range(NBUF):
            c = o * NBUF + b
            drain_gather(b)

            @pl.when(c <= NCHUNK - 1 - INFLIGHT)
            def _():
                @pl.when(c >= NBUF - INFLIGHT)
                def _():
                    drain_write((b + INFLIGHT) % NBUF)

                xform(c + INFLIGHT)
                fire(c + INFLIGHT, (b + INFLIGHT) % NBUF)

            pltpu.async_copy(
                rows_v.at[b],
                out_hbm.at[pl.ds(row_base + c * CHUNK, CHUNK)],
                wsem,
            )
        return carry

    lax.fori_loop(0, NCHUNK // NBUF, step, 0)

    for b in range(NBUF):
        drain_write(b)


def kernel(inputs, tables):
    idx = inputs.reshape(ROWS // CHUNK, CHUNK)
    tab = tables.reshape(F * V, D)
    out = _emb_lookup(idx, tab)
    return out.reshape(B, F, D)


# speedup vs baseline: 16.5596x; 1.0006x over previous
import functools

import jax
import jax.numpy as jnp
from jax import lax
from jax.experimental import pallas as pl
from jax.experimental.pallas import tpu as pltpu
from jax.experimental.pallas import tpu_sc as plsc

B = 16384
F = 26
V = 100
D = 64

NC = 2
NS = 16
NW = NC * NS

ROWS = B * F
RPW = ROWS // NW
BPC = 4
CHUNK = BPC * F
NCHUNK = RPW // CHUNK
NBUF = 8
INFLIGHT = 6

_mesh = plsc.VectorSubcoreMesh(core_axis_name="c", subcore_axis_name="s")


@functools.partial(
    pl.kernel,
    mesh=_mesh,
    out_type=jax.ShapeDtypeStruct((B, F, D), jnp.float32),
    compiler_params=pltpu.CompilerParams(use_tc_tiling_on_sc=False),
    scratch_types=[
        pltpu.VMEM((RPW,), jnp.int32),
        pltpu.VMEM((NBUF, CHUNK, D), jnp.float32),
        pltpu.VMEM_SHARED((F * V, D), jnp.float32),
        pltpu.SemaphoreType.DMA,
        pltpu.SemaphoreType.DMA,
    ],
)
def _emb_lookup(idx_hbm, tab_hbm, out_hbm, idx_v, rows_v, tab_s, gsem, wsem):
    cid = lax.axis_index("c")
    sid = lax.axis_index("s")
    wid = sid * NC + cid
    row_base = wid * RPW
    batch_base = wid * (B // NW)

    @pl.when(sid == 0)
    def _():
        pltpu.sync_copy(tab_hbm, tab_s)

    pltpu.sync_copy(idx_hbm.at[pl.ds(row_base, RPW)], idx_v)
    plsc.subcore_barrier()

    def xform_pair(p):
        for g in range(2 * CHUNK // 16):
            pos0 = p * 2 * CHUNK + g * 16
            pos = pos0 + lax.iota(jnp.int32, 16)
            f = lax.rem(pos, F)
            sl = pl.ds(pos0, 16)
            idx_v[sl] = idx_v[sl] + f * V

    def fire(c, buf):
        pltpu.async_copy(
            tab_s.at[idx_v.at[pl.ds(c * CHUNK, CHUNK)]], rows_v.at[buf], gsem
        )

    def drain_gather(buf):
        pltpu.make_async_copy(
            tab_hbm.at[pl.ds(0, CHUNK)], rows_v.at[buf], gsem
        ).wait()

    def drain_write(buf):
        pltpu.make_async_copy(
            rows_v.at[buf], tab_hbm.at[pl.ds(0, CHUNK)], wsem
        ).wait()

    for p in range((INFLIGHT + 1) // 2):
        xform_pair(p)
    for i in range(INFLIGHT):
        fire(i, i)

    def step(o, carry):
        for b in range(NBUF):
            c = o * NBUF + b
            drain_gather(b)

            @pl.when(c <= NCHUNK - 1 - INFLIGHT)
            def _():
                @pl.when(c >= NBUF - INFLIGHT)
                def _():
                    drain_write((b + INFLIGHT) % NBUF)

                if (b + INFLIGHT) % 2 == 0:
                    xform_pair((c + INFLIGHT) // 2)
                fire(c + INFLIGHT, (b + INFLIGHT) % NBUF)

            for j in range(BPC):
                pltpu.async_copy(
                    rows_v.at[b, pl.ds(j * F, F)],
                    out_hbm.at[batch_base + c * BPC + j],
                    wsem,
                )
        return carry

    lax.fori_loop(0, NCHUNK // NBUF, step, 0)

    for b in range(NBUF):
        drain_write(b)


def kernel(inputs, tables):
    idx = inputs.reshape(ROWS)
    tab = tables.reshape(F * V, D)
    return _emb_lookup(idx, tab)


# speedup vs baseline: 23.1263x; 1.3966x over previous
import functools

import jax
import jax.numpy as jnp
from jax import lax
from jax.experimental import pallas as pl
from jax.experimental.pallas import tpu as pltpu
from jax.experimental.pallas import tpu_sc as plsc

B = 16384
F = 26
V = 100
D = 64

NC = 2
NS = 16
NW = NC * NS

BT = B // 128
BTW = BT // NW
BW = 128 * BTW
NG = BW // 16

_mesh = plsc.VectorSubcoreMesh(core_axis_name="c", subcore_axis_name="s")


@functools.partial(
    pl.kernel,
    mesh=_mesh,
    out_type=jax.ShapeDtypeStruct((F, D // 8, BT, 8, 128), jnp.float32),
    compiler_params=pltpu.CompilerParams(
        use_tc_tiling_on_sc=False, needs_layout_passes=False
    ),
    scratch_types=[
        pltpu.VMEM((F, BW), jnp.int32),
        pltpu.VMEM((2, D, 128), jnp.float32),
        pltpu.VMEM((2, D // 8, BTW, 8, 128), jnp.float32),
        pltpu.VMEM_SHARED((F, D, 128), jnp.float32),
        pltpu.SemaphoreType.DMA,
        pltpu.SemaphoreType.DMA,
    ],
)
def _emb_lookup(idx_hbm, tab_hbm, out_hbm, idx_v, tf, ob, tab_s, tsem, osem):
    cid = lax.axis_index("c")
    sid = lax.axis_index("s")
    wid = sid * NC + cid

    @pl.when(sid == 0)
    def _():
        pltpu.sync_copy(tab_hbm, tab_s)

    pltpu.sync_copy(idx_hbm.at[:, pl.ds(wid * BW, BW)], idx_v)
    plsc.subcore_barrier()

    def prefetch(f, q):
        pltpu.async_copy(tab_s.at[f], tf.at[q], tsem)

    def wait_table(q):
        pltpu.make_async_copy(tab_s.at[0], tf.at[q], tsem).wait()

    def write_out(f, q):
        pltpu.async_copy(
            ob.at[q], out_hbm.at[f, :, pl.ds(wid * BTW, BTW)], osem
        )

    def wait_write(q):
        pltpu.make_async_copy(
            ob.at[q], out_hbm.at[0, :, pl.ds(wid * BTW, BTW)], osem
        ).wait()

    def compute(f, q):
        def gbody(g, carry):
            btl = g // 8
            jg = (g % 8) * 16
            idxv = idx_v[f, pl.ds(g * 16, 16)]
            for d in range(D):
                dv = jnp.full((16,), d, jnp.int32)
                val = plsc.load_gather(tf.at[q], [dv, idxv])
                ob[q, d // 8, btl, d % 8, pl.ds(jg, 16)] = val
            return carry

        lax.fori_loop(0, NG, gbody, 0)

    prefetch(0, 0)

    def step(h, carry):
        for q in range(2):
            f = h * 2 + q
            wait_table(q)

            @pl.when(f + 1 < F)
            def _():
                prefetch(f + 1, 1 - q)

            @pl.when(f >= 2)
            def _():
                wait_write(q)

            compute(f, q)
            write_out(f, q)
        return carry

    lax.fori_loop(0, F // 2, step, 0)

    for q in range(2):
        wait_write(q)


def kernel(inputs, tables):
    idx_t = inputs.T
    tab_t = jnp.pad(tables.transpose(0, 2, 1), ((0, 0), (0, 0), (0, 128 - V)))
    x = _emb_lookup(idx_t, tab_t)
    return x.transpose(2, 4, 0, 1, 3).reshape(B, F, D)


# speedup vs baseline: 67.8815x; 2.9353x over previous
import functools

import jax
import jax.numpy as jnp
from jax import lax
from jax.experimental import pallas as pl
from jax.experimental.pallas import tpu as pltpu
from jax.experimental.pallas import tpu_sc as plsc

B = 16384
F = 26
V = 100
D = 64

NC = 2
NS = 16
NW = NC * NS

BT = B // 128
BTW = BT // NW
BW = 128 * BTW
NG = BW // 16

_mesh = plsc.VectorSubcoreMesh(core_axis_name="c", subcore_axis_name="s")


@functools.partial(
    pl.kernel,
    mesh=_mesh,
    out_type=jax.ShapeDtypeStruct((F, D // 8, BT, 8, 128), jnp.float32),
    compiler_params=pltpu.CompilerParams(
        use_tc_tiling_on_sc=False, needs_layout_passes=False
    ),
    scratch_types=[
        pltpu.VMEM((F, BW), jnp.int32),
        pltpu.VMEM((2, D, 128), jnp.float32),
        pltpu.VMEM((2, D // 8, BTW, 8, 128), jnp.float32),
        pltpu.VMEM_SHARED((F, D, 128), jnp.float32),
        pltpu.SemaphoreType.DMA,
        pltpu.SemaphoreType.DMA,
    ],
)
def _emb_lookup(idx_hbm, tab_hbm, out_hbm, idx_v, tf, ob, tab_s, tsem, osem):
    cid = lax.axis_index("c")
    sid = lax.axis_index("s")
    wid = sid * NC + cid

    @pl.when(sid == 0)
    def _():
        pltpu.sync_copy(tab_hbm, tab_s)

    pltpu.sync_copy(idx_hbm.at[:, pl.ds(wid * BW, BW)], idx_v)
    plsc.subcore_barrier()

    def prefetch(f, q):
        pltpu.async_copy(tab_s.at[f], tf.at[q], tsem)

    def wait_table(q):
        pltpu.make_async_copy(tab_s.at[0], tf.at[q], tsem).wait()

    def write_out(f, q):
        pltpu.async_copy(
            ob.at[q], out_hbm.at[f, :, pl.ds(wid * BTW, BTW)], osem
        )

    def wait_write(q):
        pltpu.make_async_copy(
            ob.at[q], out_hbm.at[0, :, pl.ds(wid * BTW, BTW)], osem
        ).wait()

    def compute(f, q):
        @plsc.parallel_loop(0, NG)
        def gbody(g):
            btl = g // 8
            jg = (g % 8) * 16
            idxv = idx_v[f, pl.ds(g * 16, 16)]
            for d in range(D):
                val = plsc.load_gather(tf.at[q, d], [idxv])
                ob[q, d // 8, btl, d % 8, pl.ds(jg, 16)] = val

    prefetch(0, 0)

    def step(h, carry):
        for q in range(2):
            f = h * 2 + q
            wait_table(q)

            @pl.when(f + 1 < F)
            def _():
                prefetch(f + 1, 1 - q)

            @pl.when(f >= 2)
            def _():
                wait_write(q)

            compute(f, q)
            write_out(f, q)
        return carry

    lax.fori_loop(0, F // 2, step, 0)

    for q in range(2):
        wait_write(q)


def kernel(inputs, tables):
    idx_t = inputs.T
    tab_t = jnp.pad(tables.transpose(0, 2, 1), ((0, 0), (0, 0), (0, 128 - V)))
    x = _emb_lookup(idx_t, tab_t)
    return x.transpose(2, 4, 0, 1, 3).reshape(B, F, D)
